# baseline (speedup 1.0000x reference)
import jax
import jax.numpy as jnp
from jax import lax
from jax.experimental import pallas as pl
from jax.experimental.pallas import tpu as pltpu
from jax.experimental.pallas import tpu_sc as plsc

B = 4096
L = 200
EMB = 64
NC = 2
NS = 16
NW = NC * NS
BPW = B // NW
NGBUF = 3
NSBUF = 2


def _body(idsT_hbm, table_hbm, pos_hbm, out_hbm,
          idx_v, pos_v, gbuf, tbuf,
          gsem0, gsem1, gsem2, ssem0, ssem1):
  gsems = (gsem0, gsem1, gsem2)
  ssems = (ssem0, ssem1)
  cid = lax.axis_index("c")
  sid = lax.axis_index("s")
  wid = sid * NC + cid
  b0 = wid * BPW

  pltpu.sync_copy(idsT_hbm.at[:, pl.ds(b0, BPW)], idx_v)
  pltpu.sync_copy(pos_hbm, pos_v)

  rows_j = [lax.iota(jnp.int32, 16) + 16 * j for j in range(4)]

  def gather_start(l, gb):
    pltpu.make_async_copy(
        table_hbm.at[idx_v.at[l]], gbuf.at[gb], gsems[gb]).start()

  def gather_wait(l, gb):
    pltpu.make_async_copy(
        table_hbm.at[idx_v.at[l]], gbuf.at[gb], gsems[gb]).wait()

  def store_start(l, sb):
    pltpu.make_async_copy(
        tbuf.at[sb], out_hbm.at[l, slice(None), pl.ds(b0, BPW)],
        ssems[sb]).start()

  def store_wait(l, sb):
    pltpu.make_async_copy(
        tbuf.at[sb], out_hbm.at[l, slice(None), pl.ds(b0, BPW)],
        ssems[sb]).wait()

  def add_transpose(l, gb, sb):
    pos_slices = [pos_v[l, pl.ds(16 * j, 16)] for j in range(4)]
    def tok(r, _):
      col = jnp.full((16,), 0, jnp.int32) + r
      for j in range(4):
        v = gbuf[gb, r, pl.ds(16 * j, 16)] + pos_slices[j]
        plsc.store_scatter(tbuf.at[sb], [rows_j[j], col], v)
      return _
    lax.fori_loop(0, BPW, tok, 0, unroll=4)

  def chunk_body(l, gb, sb, with_store_wait=True):
    gather_wait(l, gb)
    if with_store_wait:
      store_wait(l - NSBUF, sb)
    add_transpose(l, gb, sb)
    store_start(l, sb)
    @pl.when(l + NGBUF < L)
    def _():
      gather_start(l + NGBUF, gb)

  for b in range(NGBUF):
    gather_start(b, b)

  for l in range(NSBUF):
    chunk_body(l, l % NGBUF, l % NSBUF, with_store_wait=False)
  for l in range(NSBUF, 6):
    chunk_body(l, l % NGBUF, l % NSBUF)

  def group(g, _):
    for i in range(6):
      l = g * 6 + i
      chunk_body(l, i % NGBUF, i % NSBUF)
    return _
  lax.fori_loop(1, 33, group, 0)

  for l in range(198, L):
    chunk_body(l, l % NGBUF, l % NSBUF)
  for l in range(L - NSBUF, L):
    store_wait(l, l % NSBUF)


@jax.jit
def _run(idsT, token_table, pos_table):
  mesh = plsc.VectorSubcoreMesh(core_axis_name="c", subcore_axis_name="s")
  f = pl.kernel(
      _body,
      out_type=jax.ShapeDtypeStruct((L, EMB, B), jnp.float32),
      mesh=mesh,
      scratch_types=[
          pltpu.VMEM((L, BPW), jnp.int32),
          pltpu.VMEM((L, EMB), jnp.float32),
          pltpu.VMEM((NGBUF, BPW, EMB), jnp.float32),
          pltpu.VMEM((NSBUF, EMB, BPW), jnp.float32),
          pltpu.SemaphoreType.DMA,
          pltpu.SemaphoreType.DMA,
          pltpu.SemaphoreType.DMA,
          pltpu.SemaphoreType.DMA,
          pltpu.SemaphoreType.DMA,
      ],
      compiler_params=pltpu.CompilerParams(
          use_tc_tiling_on_sc=False,
          disable_bounds_checks=True,
          disable_semaphore_checks=True,
          needs_layout_passes=False,
      ),
  )
  return f(idsT, token_table, pos_table)


def kernel(input_ids, token_table, pos_table):
  idsT = input_ids.T.astype(jnp.int32)
  out = _run(idsT, token_table, pos_table)
  return out.transpose(2, 0, 1)


# speedup vs baseline: 1.3629x; 1.3629x over previous
import jax
import jax.numpy as jnp
from jax import lax
from jax.experimental import pallas as pl
from jax.experimental.pallas import tpu as pltpu
from jax.experimental.pallas import tpu_sc as plsc

B = 4096
L = 200
EMB = 64
N = B * L
VOCAB_ROWS = 1000000
NC = 2
NS = 16
NW = NC * NS
PER_W = N // NW
NCHUNK = PER_W // L
SPLIT = 128
NGBUF = 3
NSBUF = 2


def _sc_body(ids_hbm, table_hbm, pos_hbm, out_hbm,
             idx_v, pos_v, gbuf, sbuf,
             gsem0, gsem1, gsem2, ssem0, ssem1):
  gsems = (gsem0, gsem1, gsem2)
  ssems = (ssem0, ssem1)
  cid = lax.axis_index("c")
  sid = lax.axis_index("s")
  wid = sid * NC + cid
  base = wid * PER_W

  pltpu.sync_copy(ids_hbm.at[pl.ds(base, PER_W)], idx_v)
  pltpu.sync_copy(pos_hbm, pos_v)

  def gather_start(c, b):
    off = c * L
    pltpu.make_async_copy(
        table_hbm.at[idx_v.at[pl.ds(off, SPLIT)]],
        gbuf.at[b, pl.ds(0, SPLIT)], gsems[b]).start()
    pltpu.make_async_copy(
        table_hbm.at[idx_v.at[pl.ds(off + SPLIT, L - SPLIT)]],
        gbuf.at[b, pl.ds(SPLIT, L - SPLIT)], gsems[b]).start()

  def gather_wait(c, b):
    off = c * L
    pltpu.make_async_copy(
        table_hbm.at[idx_v.at[pl.ds(off, SPLIT)]],
        gbuf.at[b, pl.ds(0, SPLIT)], gsems[b]).wait()
    pltpu.make_async_copy(
        table_hbm.at[idx_v.at[pl.ds(off + SPLIT, L - SPLIT)]],
        gbuf.at[b, pl.ds(SPLIT, L - SPLIT)], gsems[b]).wait()

  def store_start(c, b):
    pltpu.make_async_copy(
        sbuf.at[b], out_hbm.at[pl.ds(base + c * L, L)], ssems[b]).start()

  def store_wait(c, b):
    pltpu.make_async_copy(
        sbuf.at[b], out_hbm.at[pl.ds(base + c * L, L)], ssems[b]).wait()

  def add_pos(gb, sb):
    def row(k, _):
      r = k >> 2
      col = (k & 3) * 16
      sbuf[sb, r, pl.ds(col, 16)] = (
          gbuf[gb, r, pl.ds(col, 16)] + pos_v[r, pl.ds(col, 16)])
      return _
    lax.fori_loop(0, L * 4, row, 0, unroll=8)

  def chunk_body(c, gb, sb, with_store_wait=True, with_gather=True):
    gather_wait(c, gb)
    if with_store_wait:
      store_wait(c - NSBUF, sb)
    add_pos(gb, sb)
    store_start(c, sb)
    if with_gather:
      gather_start(c + NGBUF, gb)

  for b in range(NGBUF):
    gather_start(b, b)

  for c in range(NSBUF):
    chunk_body(c, c % NGBUF, c % NSBUF, with_store_wait=False)
  for c in range(NSBUF, 6):
    chunk_body(c, c % NGBUF, c % NSBUF)

  def group(g, _):
    for i in range(6):
      c = g * 6 + i
      chunk_body(c, i % NGBUF, i % NSBUF)
    return _
  lax.fori_loop(1, 20, group, 0)

  for c in range(120, NCHUNK):
    chunk_body(c, c % NGBUF, c % NSBUF, with_gather=(c + NGBUF < NCHUNK))
  for c in range(NCHUNK - NSBUF, NCHUNK):
    store_wait(c, c % NSBUF)


def _sc_gather(ids_flat, table_rm, pos_table):
  mesh = plsc.VectorSubcoreMesh(core_axis_name="c", subcore_axis_name="s")
  f = pl.kernel(
      _sc_body,
      out_type=jax.ShapeDtypeStruct((N, EMB), jnp.float32),
      mesh=mesh,
      scratch_types=[
          pltpu.VMEM((PER_W,), jnp.int32),
          pltpu.VMEM((L, EMB), jnp.float32),
          pltpu.VMEM((NGBUF, L, EMB), jnp.float32),
          pltpu.VMEM((NSBUF, L, EMB), jnp.float32),
          pltpu.SemaphoreType.DMA,
          pltpu.SemaphoreType.DMA,
          pltpu.SemaphoreType.DMA,
          pltpu.SemaphoreType.DMA,
          pltpu.SemaphoreType.DMA,
      ],
      compiler_params=pltpu.CompilerParams(
          use_tc_tiling_on_sc=False,
          disable_bounds_checks=True,
          disable_semaphore_checks=True,
      ),
  )
  return f(ids_flat, table_rm, pos_table)


def _tc_transpose_body(x_ref, o_ref):
  o_ref[...] = x_ref[...].T


def _tc_transpose(x, br, bc):
  r, c = x.shape
  return pl.pallas_call(
      _tc_transpose_body,
      grid=(pl.cdiv(r, br), pl.cdiv(c, bc)),
      in_specs=[pl.BlockSpec((br, bc), lambda i, j: (i, j))],
      out_specs=pl.BlockSpec((bc, br), lambda i, j: (j, i)),
      out_shape=jax.ShapeDtypeStruct((c, r), x.dtype),
  )(x)


@jax.jit
def _run(input_ids, token_table, pos_table):
  table_rm = _tc_transpose(token_table.T, 64, 8192)
  ids_flat = input_ids.reshape(-1).astype(jnp.int32)
  rows = _sc_gather(ids_flat, table_rm, pos_table)
  flat = rows.reshape(B, L * EMB)
  outp = _tc_transpose(flat, 512, 1280)
  return outp.reshape(L, EMB, B).transpose(2, 0, 1)


def kernel(input_ids, token_table, pos_table):
  return _run(input_ids, token_table, pos_table)


# speedup vs baseline: 1.3967x; 1.0248x over previous
import jax
import jax.numpy as jnp
from jax import lax
from jax.experimental import pallas as pl
from jax.experimental.pallas import tpu as pltpu
from jax.experimental.pallas import tpu_sc as plsc

B = 4096
L = 200
EMB = 64
N = B * L
VOCAB_ROWS = 1000000
NC = 2
NS = 16
NW = NC * NS
PER_W = N // NW
NCHUNK = PER_W // L
SPLIT = 128
NGBUF = 3
NSBUF = 2


def _sc_body(ids_hbm, table_hbm, pos_hbm, out_hbm,
             idx_v, pos_v, gbuf, sbuf,
             gsem0, gsem1, gsem2, ssem0, ssem1):
  gsems = (gsem0, gsem1, gsem2)
  ssems = (ssem0, ssem1)
  cid = lax.axis_index("c")
  sid = lax.axis_index("s")
  wid = sid * NC + cid
  base = wid * PER_W

  pltpu.sync_copy(ids_hbm.at[pl.ds(base, PER_W)], idx_v)
  pltpu.sync_copy(pos_hbm, pos_v)

  def gather_start(c, b):
    off = c * L
    pltpu.make_async_copy(
        table_hbm.at[idx_v.at[pl.ds(off, SPLIT)]],
        gbuf.at[b, pl.ds(0, SPLIT)], gsems[b]).start()
    pltpu.make_async_copy(
        table_hbm.at[idx_v.at[pl.ds(off + SPLIT, L - SPLIT)]],
        gbuf.at[b, pl.ds(SPLIT, L - SPLIT)], gsems[b]).start()

  def gather_wait(c, b):
    off = c * L
    pltpu.make_async_copy(
        table_hbm.at[idx_v.at[pl.ds(off, SPLIT)]],
        gbuf.at[b, pl.ds(0, SPLIT)], gsems[b]).wait()
    pltpu.make_async_copy(
        table_hbm.at[idx_v.at[pl.ds(off + SPLIT, L - SPLIT)]],
        gbuf.at[b, pl.ds(SPLIT, L - SPLIT)], gsems[b]).wait()

  def store_start(c, b):
    pltpu.make_async_copy(
        sbuf.at[b], out_hbm.at[pl.ds(base + c * L, L)], ssems[b]).start()

  def store_wait(c, b):
    pltpu.make_async_copy(
        sbuf.at[b], out_hbm.at[pl.ds(base + c * L, L)], ssems[b]).wait()

  def add_pos(gb, sb):
    def row(k, _):
      r = k >> 2
      col = (k & 3) * 16
      sbuf[sb, r, pl.ds(col, 16)] = (
          gbuf[gb, r, pl.ds(col, 16)] + pos_v[r, pl.ds(col, 16)])
      return _
    lax.fori_loop(0, L * 4, row, 0, unroll=8)

  def chunk_body(c, gb, sb, with_store_wait=True, with_gather=True):
    gather_wait(c, gb)
    if with_store_wait:
      store_wait(c - NSBUF, sb)
    add_pos(gb, sb)
    store_start(c, sb)
    if with_gather:
      gather_start(c + NGBUF, gb)

  for b in range(NGBUF):
    gather_start(b, b)

  for c in range(NSBUF):
    chunk_body(c, c % NGBUF, c % NSBUF, with_store_wait=False)
  for c in range(NSBUF, 6):
    chunk_body(c, c % NGBUF, c % NSBUF)

  def group(g, _):
    for i in range(6):
      c = g * 6 + i
      chunk_body(c, i % NGBUF, i % NSBUF)
    return _
  lax.fori_loop(1, 20, group, 0)

  for c in range(120, NCHUNK):
    chunk_body(c, c % NGBUF, c % NSBUF, with_gather=(c + NGBUF < NCHUNK))
  for c in range(NCHUNK - NSBUF, NCHUNK):
    store_wait(c, c % NSBUF)


def _sc_gather(ids_flat, table_rm, pos_table):
  mesh = plsc.VectorSubcoreMesh(core_axis_name="c", subcore_axis_name="s")
  f = pl.kernel(
      _sc_body,
      out_type=jax.ShapeDtypeStruct((N, EMB), jnp.float32),
      mesh=mesh,
      scratch_types=[
          pltpu.VMEM((PER_W,), jnp.int32),
          pltpu.VMEM((L, EMB), jnp.float32),
          pltpu.VMEM((NGBUF, L, EMB), jnp.float32),
          pltpu.VMEM((NSBUF, L, EMB), jnp.float32),
          pltpu.SemaphoreType.DMA,
          pltpu.SemaphoreType.DMA,
          pltpu.SemaphoreType.DMA,
          pltpu.SemaphoreType.DMA,
          pltpu.SemaphoreType.DMA,
      ],
      compiler_params=pltpu.CompilerParams(
          use_tc_tiling_on_sc=False,
          disable_bounds_checks=True,
          disable_semaphore_checks=True,
      ),
  )
  return f(ids_flat, table_rm, pos_table)


def _tc_transpose_body(x_ref, o_ref):
  o_ref[...] = x_ref[...].T


def _tc_transpose(x, br, bc):
  r, c = x.shape
  return pl.pallas_call(
      _tc_transpose_body,
      grid=(pl.cdiv(r, br), pl.cdiv(c, bc)),
      in_specs=[pl.BlockSpec((br, bc), lambda i, j: (i, j))],
      out_specs=pl.BlockSpec((bc, br), lambda i, j: (j, i)),
      out_shape=jax.ShapeDtypeStruct((c, r), x.dtype),
  )(x)


@jax.jit
def _run(input_ids, token_table, pos_table):
  table_rm = _tc_transpose(token_table.T, 64, 16384)
  ids_flat = input_ids.reshape(-1).astype(jnp.int32)
  rows = _sc_gather(ids_flat, table_rm, pos_table)
  flat = rows.reshape(B, L * EMB)
  outp = _tc_transpose(flat, 512, 3200)
  return outp.reshape(L, EMB, B).transpose(2, 0, 1)


def kernel(input_ids, token_table, pos_table):
  return _run(input_ids, token_table, pos_table)


# speedup vs baseline: 1.4084x; 1.0084x over previous
import jax
import jax.numpy as jnp
from jax import lax
from jax.experimental import pallas as pl
from jax.experimental.pallas import tpu as pltpu
from jax.experimental.pallas import tpu_sc as plsc

B = 4096
L = 200
EMB = 64
N = B * L
VOCAB_ROWS = 1000000
NC = 2
NS = 16
NW = NC * NS
PER_W = N // NW
NCHUNK = PER_W // L
SPLIT = 128
NGBUF = 3
NSBUF = 2


def _sc_body(ids_hbm, table_hbm, pos_hbm, out_hbm,
             idx_v, pos_v, gbuf, sbuf,
             gsem0, gsem1, gsem2, ssem0, ssem1):
  gsems = (gsem0, gsem1, gsem2)
  ssems = (ssem0, ssem1)
  cid = lax.axis_index("c")
  sid = lax.axis_index("s")
  wid = sid * NC + cid
  base = wid * PER_W

  pltpu.sync_copy(ids_hbm.at[pl.ds(base, PER_W)], idx_v)
  pltpu.sync_copy(pos_hbm, pos_v)

  def gather_start(c, b):
    off = c * L
    pltpu.make_async_copy(
        table_hbm.at[idx_v.at[pl.ds(off, SPLIT)]],
        gbuf.at[b, pl.ds(0, SPLIT)], gsems[b]).start()
    pltpu.make_async_copy(
        table_hbm.at[idx_v.at[pl.ds(off + SPLIT, L - SPLIT)]],
        gbuf.at[b, pl.ds(SPLIT, L - SPLIT)], gsems[b]).start()

  def gather_wait(c, b):
    off = c * L
    pltpu.make_async_copy(
        table_hbm.at[idx_v.at[pl.ds(off, SPLIT)]],
        gbuf.at[b, pl.ds(0, SPLIT)], gsems[b]).wait()
    pltpu.make_async_copy(
        table_hbm.at[idx_v.at[pl.ds(off + SPLIT, L - SPLIT)]],
        gbuf.at[b, pl.ds(SPLIT, L - SPLIT)], gsems[b]).wait()

  def store_start(c, b):
    pltpu.make_async_copy(
        sbuf.at[b], out_hbm.at[pl.ds(base + c * L, L)], ssems[b]).start()

  def store_wait(c, b):
    pltpu.make_async_copy(
        sbuf.at[b], out_hbm.at[pl.ds(base + c * L, L)], ssems[b]).wait()

  def add_pos(gb, sb):
    def row(k, _):
      r = k >> 2
      col = (k & 3) * 16
      sbuf[sb, r, pl.ds(col, 16)] = (
          gbuf[gb, r, pl.ds(col, 16)] + pos_v[r, pl.ds(col, 16)])
      return _
    lax.fori_loop(0, L * 4, row, 0, unroll=8)

  def chunk_body(c, gb, sb, with_store_wait=True, with_gather=True):
    gather_wait(c, gb)
    if with_store_wait:
      store_wait(c - NSBUF, sb)
    add_pos(gb, sb)
    store_start(c, sb)
    if with_gather:
      gather_start(c + NGBUF, gb)

  for b in range(NGBUF):
    gather_start(b, b)

  for c in range(NSBUF):
    chunk_body(c, c % NGBUF, c % NSBUF, with_store_wait=False)
  for c in range(NSBUF, 6):
    chunk_body(c, c % NGBUF, c % NSBUF)

  def group(g, _):
    for i in range(6):
      c = g * 6 + i
      chunk_body(c, i % NGBUF, i % NSBUF)
    return _
  lax.fori_loop(1, 20, group, 0)

  for c in range(120, NCHUNK):
    chunk_body(c, c % NGBUF, c % NSBUF, with_gather=(c + NGBUF < NCHUNK))
  for c in range(NCHUNK - NSBUF, NCHUNK):
    store_wait(c, c % NSBUF)


def _sc_gather(ids_flat, table_rm, pos_table):
  mesh = plsc.VectorSubcoreMesh(core_axis_name="c", subcore_axis_name="s")
  f = pl.kernel(
      _sc_body,
      out_type=jax.ShapeDtypeStruct((N, EMB), jnp.float32),
      mesh=mesh,
      scratch_types=[
          pltpu.VMEM((PER_W,), jnp.int32),
          pltpu.VMEM((L, EMB), jnp.float32),
          pltpu.VMEM((NGBUF, L, EMB), jnp.float32),
          pltpu.VMEM((NSBUF, L, EMB), jnp.float32),
          pltpu.SemaphoreType.DMA,
          pltpu.SemaphoreType.DMA,
          pltpu.SemaphoreType.DMA,
          pltpu.SemaphoreType.DMA,
          pltpu.SemaphoreType.DMA,
      ],
      compiler_params=pltpu.CompilerParams(
          use_tc_tiling_on_sc=False,
          disable_bounds_checks=True,
          disable_semaphore_checks=True,
      ),
  )
  return f(ids_flat, table_rm, pos_table)


def _tc_transpose_body(x_ref, o_ref):
  o_ref[...] = x_ref[...].T


def _tc_transpose(x, br, bc):
  r, c = x.shape
  return pl.pallas_call(
      _tc_transpose_body,
      grid=(pl.cdiv(r, br), pl.cdiv(c, bc)),
      in_specs=[pl.BlockSpec((br, bc), lambda i, j: (i, j))],
      out_specs=pl.BlockSpec((bc, br), lambda i, j: (j, i)),
      out_shape=jax.ShapeDtypeStruct((c, r), x.dtype),
      compiler_params=pltpu.CompilerParams(
          vmem_limit_bytes=100 * 1024 * 1024),
  )(x)


@jax.jit
def _run(input_ids, token_table, pos_table):
  table_rm = _tc_transpose(token_table.T, 64, 32768)
  ids_flat = input_ids.reshape(-1).astype(jnp.int32)
  rows = _sc_gather(ids_flat, table_rm, pos_table)
  flat = rows.reshape(B, L * EMB)
  outp = _tc_transpose(flat, 1024, 3200)
  return outp.reshape(L, EMB, B).transpose(2, 0, 1)


def kernel(input_ids, token_table, pos_table):
  return _run(input_ids, token_table, pos_table)
